# baseline (device time: 72374 ns/iter reference)
import jax
import jax.numpy as jnp
from jax import lax
from jax.experimental import pallas as pl
from jax.experimental.pallas import tpu as pltpu

N_DEV = 8


def kernel(t, W):
    m, k = t.shape
    _, n = W.shape
    ch = m // N_DEV

    def body(
        t_ref,
        w_ref,
        out_ref,
        rs_send,
        ag_send,
        rs_recv,
        ag_recv,
        rs_send_sems,
        rs_recv_sems,
        ag_send_sems,
        ag_recv_sems,
    ):
        my = lax.axis_index("i")
        left = (my - 1) % N_DEV
        right = (my + 1) % N_DEV

        barrier = pltpu.get_barrier_semaphore()
        for nbr in (left, right):
            pl.semaphore_signal(
                barrier,
                inc=1,
                device_id=(nbr,),
                device_id_type=pl.DeviceIdType.MESH,
            )
        pl.semaphore_wait(barrier, 2)

        for s in range(N_DEV - 1):
            idx = (my - s) % N_DEV
            chunk = t_ref[pl.ds(idx * ch, ch), :]
            if s == 0:
                rs_send[...] = chunk
            else:
                rs_send[...] = rs_recv[s - 1] + chunk
            rdma = pltpu.make_async_remote_copy(
                src_ref=rs_send,
                dst_ref=rs_recv.at[s],
                send_sem=rs_send_sems.at[s],
                recv_sem=rs_recv_sems.at[s],
                device_id=(right,),
                device_id_type=pl.DeviceIdType.MESH,
            )
            rdma.start()
            rdma.wait()

        own_idx = (my + 1) % N_DEV
        owned = rs_recv[N_DEV - 2] + t_ref[pl.ds(own_idx * ch, ch), :]

        result = jnp.dot(owned, w_ref[...], preferred_element_type=jnp.float32)
        ag_send[...] = result
        out_ref[pl.ds(own_idx * ch, ch), :] = result

        for s in range(N_DEV - 1):
            src = ag_send if s == 0 else ag_recv.at[s - 1]
            rdma = pltpu.make_async_remote_copy(
                src_ref=src,
                dst_ref=ag_recv.at[s],
                send_sem=ag_send_sems.at[s],
                recv_sem=ag_recv_sems.at[s],
                device_id=(right,),
                device_id_type=pl.DeviceIdType.MESH,
            )
            rdma.start()
            rdma.wait()
            origin = (my - s) % N_DEV
            out_ref[pl.ds(origin * ch, ch), :] = ag_recv[s]

    return pl.pallas_call(
        body,
        out_shape=jax.ShapeDtypeStruct((m, n), jnp.float32),
        in_specs=[
            pl.BlockSpec(memory_space=pltpu.VMEM),
            pl.BlockSpec(memory_space=pltpu.VMEM),
        ],
        out_specs=pl.BlockSpec(memory_space=pltpu.VMEM),
        scratch_shapes=[
            pltpu.VMEM((ch, k), jnp.float32),
            pltpu.VMEM((ch, n), jnp.float32),
            pltpu.VMEM((N_DEV - 1, ch, k), jnp.float32),
            pltpu.VMEM((N_DEV - 1, ch, n), jnp.float32),
            pltpu.SemaphoreType.DMA((N_DEV - 1,)),
            pltpu.SemaphoreType.DMA((N_DEV - 1,)),
            pltpu.SemaphoreType.DMA((N_DEV - 1,)),
            pltpu.SemaphoreType.DMA((N_DEV - 1,)),
        ],
        compiler_params=pltpu.CompilerParams(collective_id=0),
    )(t, W)


# device time: 56973 ns/iter; 1.2703x vs baseline; 1.2703x over previous
import jax
import jax.numpy as jnp
from jax import lax
from jax.experimental import pallas as pl
from jax.experimental.pallas import tpu as pltpu

N_DEV = 8


def kernel(t, W):
    m, k = t.shape
    _, n = W.shape
    ch = m // N_DEV

    def body(
        t_ref,
        w_ref,
        out_ref,
        acc1,
        acc2,
        recv1,
        recv2,
        recv3,
        send_sems,
        recv_sems,
    ):
        pos = lax.axis_index("i")
        b0 = pos & 1
        b1 = (pos >> 1) & 1
        b2 = (pos >> 2) & 1
        x = b0 ^ b1
        y = b1
        z = b2
        px = pos ^ 1
        py = pos ^ 3
        pz = pos ^ 4

        barrier = pltpu.get_barrier_semaphore()
        for nbr in (px, py, pz):
            pl.semaphore_signal(
                barrier,
                inc=1,
                device_id=(nbr,),
                device_id_type=pl.DeviceIdType.MESH,
            )
        pl.semaphore_wait(barrier, 3)

        r1 = pltpu.make_async_remote_copy(
            src_ref=t_ref.at[pl.ds((1 - x) * (4 * ch), 4 * ch)],
            dst_ref=recv1,
            send_sem=send_sems.at[0],
            recv_sem=recv_sems.at[0],
            device_id=(px,),
            device_id_type=pl.DeviceIdType.MESH,
        )
        r1.start()
        r1.wait()
        acc1[...] = t_ref[pl.ds(x * (4 * ch), 4 * ch), :] + recv1[...]

        r2 = pltpu.make_async_remote_copy(
            src_ref=acc1.at[pl.ds((1 - y) * (2 * ch), 2 * ch)],
            dst_ref=recv2,
            send_sem=send_sems.at[1],
            recv_sem=recv_sems.at[1],
            device_id=(py,),
            device_id_type=pl.DeviceIdType.MESH,
        )
        r2.start()
        r2.wait()
        acc2[...] = acc1[pl.ds(y * (2 * ch), 2 * ch), :] + recv2[...]

        r3 = pltpu.make_async_remote_copy(
            src_ref=acc2.at[pl.ds((1 - z) * ch, ch)],
            dst_ref=recv3,
            send_sem=send_sems.at[2],
            recv_sem=recv_sems.at[2],
            device_id=(pz,),
            device_id_type=pl.DeviceIdType.MESH,
        )
        r3.start()
        r3.wait()
        owned = acc2[pl.ds(z * ch, ch), :] + recv3[...]

        result = jnp.dot(owned, w_ref[...], preferred_element_type=jnp.float32)
        c_me = 4 * x + 2 * y + z
        out_ref[pl.ds(c_me * ch, ch), :] = result

        a1 = pltpu.make_async_remote_copy(
            src_ref=out_ref.at[pl.ds(c_me * ch, ch)],
            dst_ref=out_ref.at[pl.ds(c_me * ch, ch)],
            send_sem=send_sems.at[3],
            recv_sem=recv_sems.at[3],
            device_id=(pz,),
            device_id_type=pl.DeviceIdType.MESH,
        )
        a1.start()
        a1.wait()
        base2 = (4 * x + 2 * y) * ch
        a2 = pltpu.make_async_remote_copy(
            src_ref=out_ref.at[pl.ds(base2, 2 * ch)],
            dst_ref=out_ref.at[pl.ds(base2, 2 * ch)],
            send_sem=send_sems.at[4],
            recv_sem=recv_sems.at[4],
            device_id=(py,),
            device_id_type=pl.DeviceIdType.MESH,
        )
        a2.start()
        a2.wait()
        base1 = x * (4 * ch)
        a3 = pltpu.make_async_remote_copy(
            src_ref=out_ref.at[pl.ds(base1, 4 * ch)],
            dst_ref=out_ref.at[pl.ds(base1, 4 * ch)],
            send_sem=send_sems.at[5],
            recv_sem=recv_sems.at[5],
            device_id=(px,),
            device_id_type=pl.DeviceIdType.MESH,
        )
        a3.start()
        a3.wait()

    return pl.pallas_call(
        body,
        out_shape=jax.ShapeDtypeStruct((m, n), jnp.float32),
        in_specs=[
            pl.BlockSpec(memory_space=pltpu.VMEM),
            pl.BlockSpec(memory_space=pltpu.VMEM),
        ],
        out_specs=pl.BlockSpec(memory_space=pltpu.VMEM),
        scratch_shapes=[
            pltpu.VMEM((4 * ch, k), jnp.float32),
            pltpu.VMEM((2 * ch, k), jnp.float32),
            pltpu.VMEM((4 * ch, k), jnp.float32),
            pltpu.VMEM((2 * ch, k), jnp.float32),
            pltpu.VMEM((ch, k), jnp.float32),
            pltpu.SemaphoreType.DMA((6,)),
            pltpu.SemaphoreType.DMA((6,)),
        ],
        compiler_params=pltpu.CompilerParams(collective_id=0),
    )(t, W)


# device time: 38190 ns/iter; 1.8951x vs baseline; 1.4918x over previous
import jax
import jax.numpy as jnp
from jax import lax
from jax.experimental import pallas as pl
from jax.experimental.pallas import tpu as pltpu

N_DEV = 8
SEND_ORDER = [6, 2, 5, 7, 1, 3, 4]
WAIT_ORDER = [1, 3, 4, 2, 5, 7, 6]


def kernel(t, W):
    m, k = t.shape
    _, n = W.shape
    ch = m // N_DEV

    def chunk_of(p):
        b0 = p & 1
        b1 = (p >> 1) & 1
        b2 = (p >> 2) & 1
        return 4 * (b0 ^ b1) + 2 * b1 + b2

    def body(
        t_ref,
        w_ref,
        out_ref,
        rs_recv,
        rs_send_sems,
        rs_recv_sems,
        ag_send_sems,
        ag_recv_sems,
    ):
        pos = lax.axis_index("i")
        c_me = chunk_of(pos)

        barrier = pltpu.get_barrier_semaphore()
        for mask in range(1, N_DEV):
            pl.semaphore_signal(
                barrier,
                inc=1,
                device_id=(pos ^ mask,),
                device_id_type=pl.DeviceIdType.MESH,
            )
        pl.semaphore_wait(barrier, N_DEV - 1)

        def exchange(mask, src, dst, send_sems, recv_sems):
            return pltpu.make_async_remote_copy(
                src_ref=src,
                dst_ref=dst,
                send_sem=send_sems.at[mask - 1],
                recv_sem=recv_sems.at[mask - 1],
                device_id=(pos ^ mask,),
                device_id_type=pl.DeviceIdType.MESH,
            )

        sends = []
        for mask in SEND_ORDER:
            c_q = chunk_of(pos ^ mask)
            r = exchange(
                mask,
                t_ref.at[pl.ds(c_q * ch, ch)],
                rs_recv.at[mask - 1],
                rs_send_sems,
                rs_recv_sems,
            )
            r.start()
            sends.append(r)

        acc = t_ref[pl.ds(c_me * ch, ch), :]
        for mask in WAIT_ORDER:
            rw = exchange(
                mask,
                rs_recv.at[mask - 1],
                rs_recv.at[mask - 1],
                rs_send_sems,
                rs_recv_sems,
            )
            rw.wait_recv()
            acc = acc + rs_recv[mask - 1]

        result = jnp.dot(acc, w_ref[...], preferred_element_type=jnp.float32)
        out_ref[pl.ds(c_me * ch, ch), :] = result

        for mask in SEND_ORDER:
            a = exchange(
                mask,
                out_ref.at[pl.ds(c_me * ch, ch)],
                out_ref.at[pl.ds(c_me * ch, ch)],
                ag_send_sems,
                ag_recv_sems,
            )
            a.start()
            sends.append(a)

        for mask in WAIT_ORDER:
            c_p = chunk_of(pos ^ mask)
            aw = exchange(
                mask,
                out_ref.at[pl.ds(c_p * ch, ch)],
                out_ref.at[pl.ds(c_p * ch, ch)],
                ag_send_sems,
                ag_recv_sems,
            )
            aw.wait_recv()

        for r in sends:
            r.wait_send()

    return pl.pallas_call(
        body,
        out_shape=jax.ShapeDtypeStruct((m, n), jnp.float32),
        in_specs=[
            pl.BlockSpec(memory_space=pltpu.VMEM),
            pl.BlockSpec(memory_space=pltpu.VMEM),
        ],
        out_specs=pl.BlockSpec(memory_space=pltpu.VMEM),
        scratch_shapes=[
            pltpu.VMEM((N_DEV - 1, ch, k), jnp.float32),
            pltpu.SemaphoreType.DMA((N_DEV - 1,)),
            pltpu.SemaphoreType.DMA((N_DEV - 1,)),
            pltpu.SemaphoreType.DMA((N_DEV - 1,)),
            pltpu.SemaphoreType.DMA((N_DEV - 1,)),
        ],
        compiler_params=pltpu.CompilerParams(collective_id=0),
    )(t, W)
